# Initial kernel scaffold; baseline (speedup 1.0000x reference)
import jax
import jax.numpy as jnp
from jax.experimental import pallas as pl


def kernel(x, edge_index, W1, b1, W2, b2):
    raise NotImplementedError("write your pallas kernel here")


# speedup vs baseline: 2.6083x; 2.6083x over previous
import functools

import jax
import jax.numpy as jnp
from jax import lax
from jax.experimental import pallas as pl
from jax.experimental.pallas import tpu as pltpu
from jax.experimental.pallas import tpu_sc as plsc

D = 128
_NC = 2
_NS = 16
_NW = _NC * _NS


def _mlp1_body(x_ref, w1a_ref, w1b_ref, b1_ref, y1_ref, y2_ref):
    xb = x_ref[...]
    y1_ref[...] = (
        jnp.dot(xb, w1a_ref[...], preferred_element_type=jnp.float32)
        + b1_ref[...]
    )
    y2_ref[...] = jnp.dot(xb, w1b_ref[...], preferred_element_type=jnp.float32)


def _node_tables(x, W1, b1):
    n, d = x.shape
    bn = 1000
    y1, y2 = pl.pallas_call(
        _mlp1_body,
        grid=(n // bn,),
        in_specs=[
            pl.BlockSpec((bn, d), lambda i: (i, 0)),
            pl.BlockSpec((d, d), lambda i: (0, 0)),
            pl.BlockSpec((d, d), lambda i: (0, 0)),
            pl.BlockSpec((1, d), lambda i: (0, 0)),
        ],
        out_specs=[
            pl.BlockSpec((bn, d), lambda i: (i, 0)),
            pl.BlockSpec((bn, d), lambda i: (i, 0)),
        ],
        out_shape=[
            jax.ShapeDtypeStruct((n, d), jnp.float32),
            jax.ShapeDtypeStruct((n, d), jnp.float32),
        ],
    )(x, W1[:d], W1[d:], b1.reshape(1, d))
    return y1, y2


def _make_sc_kernel(E, C):
    epw = E // _NW
    n_chunks = epw // C
    mesh = plsc.VectorSubcoreMesh(core_axis_name="c", subcore_axis_name="s")

    @functools.partial(
        pl.kernel,
        mesh=mesh,
        out_type=jax.ShapeDtypeStruct((E, 16), jnp.float32),
        scratch_types=[
            pltpu.VMEM((C,), jnp.int32),
            pltpu.VMEM((C,), jnp.int32),
            pltpu.VMEM((C, D), jnp.float32),
            pltpu.VMEM((C, D), jnp.float32),
            pltpu.VMEM((C, 16), jnp.float32),
            pltpu.VMEM((D,), jnp.float32),
            pltpu.SemaphoreType.DMA,
            pltpu.SemaphoreType.DMA,
        ],
    )
    def sc_edge_mlp(y1_hbm, y2_hbm, src_hbm, dst_hbm, w2_hbm, out_hbm,
                    src_v, dst_v, rows1, rows2, outc, w2_v, sem1, sem2):
        wid = lax.axis_index("s") * _NC + lax.axis_index("c")
        base = wid * epw
        pltpu.sync_copy(w2_hbm, w2_v)

        def chunk(i, carry):
            off = base + i * C
            pltpu.sync_copy(src_hbm.at[pl.ds(off, C)], src_v)
            pltpu.sync_copy(dst_hbm.at[pl.ds(off, C)], dst_v)
            cp1 = pltpu.async_copy(y1_hbm.at[src_v], rows1, sem1)
            cp2 = pltpu.async_copy(y2_hbm.at[dst_v], rows2, sem2)
            cp1.wait()
            cp2.wait()

            def edge(e, c2):
                acc = jnp.zeros((16,), jnp.float32)
                for k in range(D // 16):
                    s = k * 16
                    a = rows1[e, pl.ds(s, 16)]
                    b = rows2[e, pl.ds(s, 16)]
                    z = jnp.maximum(a + b, 0.0)
                    acc = acc + z * w2_v[pl.ds(s, 16)]
                outc[e] = acc
                return c2

            lax.fori_loop(0, C, edge, 0)
            pltpu.sync_copy(outc, out_hbm.at[pl.ds(off, C)])
            return carry

        lax.fori_loop(0, n_chunks, chunk, 0)

    return sc_edge_mlp


def _lane_sum_body(p_ref, o_ref):
    i = lax.broadcasted_iota(jnp.int32, (D, 8), 0)
    j = lax.broadcasted_iota(jnp.int32, (D, 8), 1)
    m = (i // 16 == j).astype(jnp.float32)
    o_ref[...] = jnp.dot(p_ref[...], m, preferred_element_type=jnp.float32)


def _lane_sum(partials, E):
    rows = E // 8
    br = 4000
    p2 = partials.reshape(rows, D)
    out = pl.pallas_call(
        _lane_sum_body,
        grid=(rows // br,),
        in_specs=[pl.BlockSpec((br, D), lambda i: (i, 0))],
        out_specs=pl.BlockSpec((br, 8), lambda i: (i, 0)),
        out_shape=jax.ShapeDtypeStruct((rows, 8), jnp.float32),
    )(p2)
    return out.reshape(E)


def kernel(x, edge_index, W1, b1, W2, b2):
    y1, y2 = _node_tables(x, W1, b1)
    src = edge_index[0]
    dst = edge_index[1]
    E = src.shape[0]
    sc = _make_sc_kernel(E, C=80)
    partials = sc(y1, y2, src, dst, W2.reshape(-1))
    return _lane_sum(partials, E) + b2[0]


# speedup vs baseline: 3.4731x; 1.3315x over previous
import functools

import jax
import jax.numpy as jnp
from jax import lax
from jax.experimental import pallas as pl
from jax.experimental.pallas import tpu as pltpu
from jax.experimental.pallas import tpu_sc as plsc

D = 128
_NC = 2
_NS = 16
_NW = _NC * _NS


def _mlp1_body(x_ref, w1a_ref, w1b_ref, b1_ref, y1_ref, y2_ref):
    xb = x_ref[...]
    y1_ref[...] = (
        jnp.dot(xb, w1a_ref[...], preferred_element_type=jnp.float32)
        + b1_ref[...]
    )
    y2_ref[...] = jnp.dot(xb, w1b_ref[...], preferred_element_type=jnp.float32)


def _node_tables(x, W1, b1):
    n, d = x.shape
    bn = 1000
    y1, y2 = pl.pallas_call(
        _mlp1_body,
        grid=(n // bn,),
        in_specs=[
            pl.BlockSpec((bn, d), lambda i: (i, 0)),
            pl.BlockSpec((d, d), lambda i: (0, 0)),
            pl.BlockSpec((d, d), lambda i: (0, 0)),
            pl.BlockSpec((1, d), lambda i: (0, 0)),
        ],
        out_specs=[
            pl.BlockSpec((bn, d), lambda i: (i, 0)),
            pl.BlockSpec((bn, d), lambda i: (i, 0)),
        ],
        out_shape=[
            jax.ShapeDtypeStruct((n, d), jnp.float32),
            jax.ShapeDtypeStruct((n, d), jnp.float32),
        ],
    )(x, W1[:d], W1[d:], b1.reshape(1, d))
    return y1, y2


def _make_sc_kernel(E, C):
    epw = E // _NW
    n_chunks = epw // C
    assert n_chunks % 2 == 0 and epw % C == 0 and C % 8 == 0
    mesh = plsc.VectorSubcoreMesh(core_axis_name="c", subcore_axis_name="s")

    @functools.partial(
        pl.kernel,
        mesh=mesh,
        out_type=jax.ShapeDtypeStruct((E, 16), jnp.float32),
        scratch_types=[
            pltpu.VMEM((2, C), jnp.int32),
            pltpu.VMEM((2, C), jnp.int32),
            pltpu.VMEM((2, C, D), jnp.float32),
            pltpu.VMEM((2, C, D), jnp.float32),
            pltpu.VMEM((2, C, 16), jnp.float32),
            pltpu.VMEM((D,), jnp.float32),
            pltpu.SemaphoreType.DMA,
            pltpu.SemaphoreType.DMA,
            pltpu.SemaphoreType.DMA,
            pltpu.SemaphoreType.DMA,
            pltpu.SemaphoreType.DMA,
            pltpu.SemaphoreType.DMA,
        ],
    )
    def sc_edge_mlp(y1_hbm, y2_hbm, src_hbm, dst_hbm, w2_hbm, out_hbm,
                    src_v, dst_v, rows1, rows2, outc, w2_v,
                    sem_i0, sem_i1, sem_g0, sem_g1, sem_o0, sem_o1):
        sem_i = (sem_i0, sem_i1)
        sem_g = (sem_g0, sem_g1)
        sem_o = (sem_o0, sem_o1)
        wid = lax.axis_index("s") * _NC + lax.axis_index("c")
        base = wid * epw
        pltpu.sync_copy(w2_hbm, w2_v)

        def issue_idx(c, b):
            off = base + c * C
            pltpu.async_copy(src_hbm.at[pl.ds(off, C)], src_v.at[b], sem_i[b])
            pltpu.async_copy(dst_hbm.at[pl.ds(off, C)], dst_v.at[b], sem_i[b])

        def wait_idx(b):
            pltpu.make_async_copy(src_hbm.at[pl.ds(0, C)], src_v.at[b],
                                  sem_i[b]).wait()
            pltpu.make_async_copy(dst_hbm.at[pl.ds(0, C)], dst_v.at[b],
                                  sem_i[b]).wait()

        def issue_gather(b):
            pltpu.async_copy(y1_hbm.at[src_v.at[b]], rows1.at[b], sem_g[b])
            pltpu.async_copy(y2_hbm.at[dst_v.at[b]], rows2.at[b], sem_g[b])

        def wait_gather(b):
            pltpu.make_async_copy(y1_hbm.at[src_v.at[b]], rows1.at[b],
                                  sem_g[b]).wait()
            pltpu.make_async_copy(y2_hbm.at[dst_v.at[b]], rows2.at[b],
                                  sem_g[b]).wait()

        def issue_store(c, b):
            off = base + c * C
            pltpu.async_copy(outc.at[b], out_hbm.at[pl.ds(off, C)], sem_o[b])

        def wait_store(b):
            pltpu.make_async_copy(outc.at[b], out_hbm.at[pl.ds(0, C)],
                                  sem_o[b]).wait()

        issue_idx(0, 0)
        issue_idx(1, 1)
        wait_idx(0)
        issue_gather(0)

        def step(si, carry):
            for b in (0, 1):
                s = 2 * si + b
                nb = 1 - b
                wait_gather(b)

                @pl.when(s + 2 < n_chunks)
                def _():
                    issue_idx(s + 2, b)

                @pl.when(s + 1 < n_chunks)
                def _():
                    wait_idx(nb)
                    issue_gather(nb)

                @pl.when(s >= 2)
                def _():
                    wait_store(b)

                def edge(e, c2):
                    acc = jnp.zeros((16,), jnp.float32)
                    for k in range(D // 16):
                        f = k * 16
                        av = rows1[b, e, pl.ds(f, 16)]
                        bv = rows2[b, e, pl.ds(f, 16)]
                        z = jnp.maximum(av + bv, 0.0)
                        acc = acc + z * w2_v[pl.ds(f, 16)]
                    outc[b, e] = acc
                    return c2

                lax.fori_loop(0, C, edge, 0)
                issue_store(s, b)
            return carry

        lax.fori_loop(0, n_chunks // 2, step, 0)
        wait_store(0)
        wait_store(1)

    return sc_edge_mlp


def _lane_sum_body(p_ref, o_ref):
    i = lax.broadcasted_iota(jnp.int32, (D, 8), 0)
    j = lax.broadcasted_iota(jnp.int32, (D, 8), 1)
    m = (i // 16 == j).astype(jnp.float32)
    o_ref[...] = jnp.dot(p_ref[...], m, preferred_element_type=jnp.float32)


def _lane_sum(partials, E):
    rows = E // 8
    br = 4000
    p2 = partials.reshape(rows, D)
    out = pl.pallas_call(
        _lane_sum_body,
        grid=(rows // br,),
        in_specs=[pl.BlockSpec((br, D), lambda i: (i, 0))],
        out_specs=pl.BlockSpec((br, 8), lambda i: (i, 0)),
        out_shape=jax.ShapeDtypeStruct((rows, 8), jnp.float32),
    )(p2)
    return out.reshape(E)


def kernel(x, edge_index, W1, b1, W2, b2):
    y1, y2 = _node_tables(x, W1, b1)
    src = edge_index[0]
    dst = edge_index[1]
    E = src.shape[0]
    sc = _make_sc_kernel(E, C=40)
    partials = sc(y1, y2, src, dst, W2.reshape(-1))
    return _lane_sum(partials, E) + b2[0]


# speedup vs baseline: 4.6657x; 1.3434x over previous
import functools

import jax
import jax.numpy as jnp
from jax import lax
from jax.experimental import pallas as pl
from jax.experimental.pallas import tpu as pltpu
from jax.experimental.pallas import tpu_sc as plsc

D = 128
_NC = 2
_NS = 16
_NW = _NC * _NS


def _mlp1_body(x_ref, w1a_ref, w1b_ref, b1_ref, y1_ref, y2_ref):
    xb = x_ref[...]
    y1_ref[...] = (
        jnp.dot(xb, w1a_ref[...], preferred_element_type=jnp.float32)
        + b1_ref[...]
    )
    y2_ref[...] = jnp.dot(xb, w1b_ref[...], preferred_element_type=jnp.float32)


def _node_tables(x, W1, b1):
    n, d = x.shape
    bn = 1000
    y1, y2 = pl.pallas_call(
        _mlp1_body,
        grid=(n // bn,),
        in_specs=[
            pl.BlockSpec((bn, d), lambda i: (i, 0)),
            pl.BlockSpec((d, d), lambda i: (0, 0)),
            pl.BlockSpec((d, d), lambda i: (0, 0)),
            pl.BlockSpec((1, d), lambda i: (0, 0)),
        ],
        out_specs=[
            pl.BlockSpec((bn, d), lambda i: (i, 0)),
            pl.BlockSpec((bn, d), lambda i: (i, 0)),
        ],
        out_shape=[
            jax.ShapeDtypeStruct((n, d), jnp.float32),
            jax.ShapeDtypeStruct((n, d), jnp.float32),
        ],
    )(x, W1[:d], W1[d:], b1.reshape(1, d))
    return y1, y2


def _make_sc_kernel(E, C):
    epw = E // _NW
    n_chunks = epw // C
    assert n_chunks % 2 == 0 and epw % C == 0 and C % 8 == 0
    mesh = plsc.VectorSubcoreMesh(core_axis_name="c", subcore_axis_name="s")

    @functools.partial(
        pl.kernel,
        mesh=mesh,
        out_type=jax.ShapeDtypeStruct((E * 16,), jnp.float32),
        scratch_types=[
            pltpu.VMEM((2, C), jnp.int32),
            pltpu.VMEM((2, C), jnp.int32),
            pltpu.VMEM((2, C, D), jnp.float32),
            pltpu.VMEM((2, C, D), jnp.float32),
            pltpu.VMEM((2, C * 16), jnp.float32),
            pltpu.VMEM((D,), jnp.float32),
            pltpu.SemaphoreType.DMA,
            pltpu.SemaphoreType.DMA,
            pltpu.SemaphoreType.DMA,
            pltpu.SemaphoreType.DMA,
            pltpu.SemaphoreType.DMA,
            pltpu.SemaphoreType.DMA,
        ],
    )
    def sc_edge_mlp(y1_hbm, y2_hbm, src_hbm, dst_hbm, w2_hbm, out_hbm,
                    src_v, dst_v, rows1, rows2, outc, w2_v,
                    sem_i0, sem_i1, sem_g0, sem_g1, sem_o0, sem_o1):
        sem_i = (sem_i0, sem_i1)
        sem_g = (sem_g0, sem_g1)
        sem_o = (sem_o0, sem_o1)
        wid = lax.axis_index("s") * _NC + lax.axis_index("c")
        base = wid * epw
        pltpu.sync_copy(w2_hbm, w2_v)

        def issue_idx(c, b):
            off = base + c * C
            pltpu.async_copy(src_hbm.at[pl.ds(off, C)], src_v.at[b], sem_i[b])
            pltpu.async_copy(dst_hbm.at[pl.ds(off, C)], dst_v.at[b], sem_i[b])

        def wait_idx(b):
            pltpu.make_async_copy(src_hbm.at[pl.ds(0, C)], src_v.at[b],
                                  sem_i[b]).wait()
            pltpu.make_async_copy(dst_hbm.at[pl.ds(0, C)], dst_v.at[b],
                                  sem_i[b]).wait()

        def issue_gather(b):
            pltpu.async_copy(y1_hbm.at[src_v.at[b]], rows1.at[b], sem_g[b])
            pltpu.async_copy(y2_hbm.at[dst_v.at[b]], rows2.at[b], sem_g[b])

        def wait_gather(b):
            pltpu.make_async_copy(y1_hbm.at[src_v.at[b]], rows1.at[b],
                                  sem_g[b]).wait()
            pltpu.make_async_copy(y2_hbm.at[dst_v.at[b]], rows2.at[b],
                                  sem_g[b]).wait()

        def issue_store(c, b):
            off = (base + c * C) * 16
            pltpu.async_copy(outc.at[b], out_hbm.at[pl.ds(off, C * 16)],
                             sem_o[b])

        def wait_store(b):
            pltpu.make_async_copy(outc.at[b], out_hbm.at[pl.ds(0, C * 16)],
                                  sem_o[b]).wait()

        issue_idx(0, 0)
        issue_idx(1, 1)
        wait_idx(0)
        issue_gather(0)

        def step(si, carry):
            for b in (0, 1):
                s = 2 * si + b
                nb = 1 - b
                wait_gather(b)

                @pl.when(s + 2 < n_chunks)
                def _():
                    issue_idx(s + 2, b)

                @pl.when(s + 1 < n_chunks)
                def _():
                    wait_idx(nb)
                    issue_gather(nb)

                @pl.when(s >= 2)
                def _():
                    wait_store(b)

                def edge(e, c2):
                    acc = jnp.zeros((16,), jnp.float32)
                    for k in range(D // 16):
                        f = k * 16
                        av = rows1[b, e, pl.ds(f, 16)]
                        bv = rows2[b, e, pl.ds(f, 16)]
                        z = jnp.maximum(av + bv, 0.0)
                        acc = acc + z * w2_v[pl.ds(f, 16)]
                    outc[b, pl.ds(e * 16, 16)] = acc
                    return c2

                lax.fori_loop(0, C, edge, 0)
                issue_store(s, b)
            return carry

        lax.fori_loop(0, n_chunks // 2, step, 0)
        wait_store(0)
        wait_store(1)

    return sc_edge_mlp


def _lane_sum_body(p_ref, o_ref):
    i = lax.broadcasted_iota(jnp.int32, (D, 8), 0)
    j = lax.broadcasted_iota(jnp.int32, (D, 8), 1)
    m = (i // 16 == j).astype(jnp.float32)
    o_ref[...] = jnp.dot(p_ref[...], m, preferred_element_type=jnp.float32)


def _lane_sum(partials, E):
    rows = E // 8
    br = 4000
    p2 = partials.reshape(rows, D)
    out = pl.pallas_call(
        _lane_sum_body,
        grid=(rows // br,),
        in_specs=[pl.BlockSpec((br, D), lambda i: (i, 0))],
        out_specs=pl.BlockSpec((br, 8), lambda i: (i, 0)),
        out_shape=jax.ShapeDtypeStruct((rows, 8), jnp.float32),
    )(p2)
    return out.reshape(E)


def kernel(x, edge_index, W1, b1, W2, b2):
    y1, y2 = _node_tables(x, W1, b1)
    src = edge_index[0]
    dst = edge_index[1]
    E = src.shape[0]
    sc = _make_sc_kernel(E, C=40)
    partials = sc(y1, y2, src, dst, W2.reshape(-1))
    return _lane_sum(partials, E) + b2[0]
